# Initial kernel scaffold; baseline (speedup 1.0000x reference)
import jax
import jax.numpy as jnp
from jax.experimental import pallas as pl


def kernel(adj, features, W1, b1, W2, b2, W3, b3, attn_in_w, attn_in_b, attn_out_w, attn_out_b, W4, b4, W5a, b5a, W6a, b6a, W7a, b7a, W5f, b5f, W6f, b6f, W7f, b7f):
    raise NotImplementedError("write your pallas kernel here")


# speedup vs baseline: 17.4152x; 17.4152x over previous
import functools

import jax
import jax.numpy as jnp
from jax import lax
from jax.experimental import pallas as pl
from jax.experimental.pallas import tpu as pltpu
from jax.experimental.pallas import tpu_sc as plsc

N = 10000
N_PAD = 10240
E = 320000
E_PAD = 327680
ROWS128 = E_PAD // 128
NC, NS = 2, 16
NW = NC * NS
CHUNKS_PER_W = ROWS128 // NW


def _sc_edge_scatter(h, src2d, dst2d, zero_rows, D, K):
    mesh = plsc.VectorSubcoreMesh(core_axis_name="c", subcore_axis_name="s")
    n_iter = CHUNKS_PER_W // K
    rows_per_sub = N_PAD // NS

    @functools.partial(
        pl.kernel,
        out_type=jax.ShapeDtypeStruct((NC, N_PAD, D), jnp.float32),
        mesh=mesh,
        scratch_types=[
            pltpu.VMEM((K, 128), jnp.int32),
            pltpu.VMEM((K, 128), jnp.int32),
            pltpu.VMEM((K, 128, D), jnp.float32),
            pltpu.VMEM_SHARED((N_PAD, D), jnp.float32),
            pltpu.SemaphoreType.DMA,
        ],
        compiler_params=pltpu.CompilerParams(use_tc_tiling_on_sc=False),
    )
    def k(h_hbm, src_hbm, dst_hbm, zero_hbm, out_hbm, src_v, dst_v, rows_v, acc_sh, sem):
        cid = lax.axis_index("c")
        sid = lax.axis_index("s")
        wid = sid * NC + cid
        pltpu.sync_copy(zero_hbm.at[pl.ds(sid * rows_per_sub, rows_per_sub)],
                        acc_sh.at[pl.ds(sid * rows_per_sub, rows_per_sub)])
        plsc.subcore_barrier()

        def body(i, carry):
            base = wid * CHUNKS_PER_W + i * K
            pltpu.sync_copy(src_hbm.at[pl.ds(base, K)], src_v)
            pltpu.sync_copy(dst_hbm.at[pl.ds(base, K)], dst_v)
            cps = [pltpu.async_copy(h_hbm.at[src_v.at[j]], rows_v.at[j], sem)
                   for j in range(K)]
            for cp in cps:
                cp.wait()
            for j in range(K):
                pltpu.sync_copy(rows_v.at[j], acc_sh.at[dst_v.at[j]], add=True)
            return carry

        lax.fori_loop(0, n_iter, body, 0)
        plsc.subcore_barrier()
        pltpu.sync_copy(acc_sh.at[pl.ds(sid * rows_per_sub, rows_per_sub)],
                        out_hbm.at[cid].at[pl.ds(sid * rows_per_sub, rows_per_sub)])

    return k(h, src2d, dst2d, zero_rows)


def _sc_degree(dst2d, zero_col, ones128):
    mesh = plsc.VectorSubcoreMesh(core_axis_name="c", subcore_axis_name="s")
    K = 8
    n_iter = CHUNKS_PER_W // K
    rows_per_sub = N_PAD // NS

    @functools.partial(
        pl.kernel,
        out_type=jax.ShapeDtypeStruct((NC, N_PAD, 1), jnp.float32),
        mesh=mesh,
        scratch_types=[
            pltpu.VMEM((K, 128), jnp.int32),
            pltpu.VMEM((128, 1), jnp.float32),
            pltpu.VMEM_SHARED((N_PAD, 1), jnp.float32),
        ],
        compiler_params=pltpu.CompilerParams(use_tc_tiling_on_sc=False),
    )
    def k(dst_hbm, zero_hbm, ones_hbm, out_hbm, dst_v, ones_v, acc_sh):
        cid = lax.axis_index("c")
        sid = lax.axis_index("s")
        wid = sid * NC + cid
        pltpu.sync_copy(ones_hbm, ones_v)
        pltpu.sync_copy(zero_hbm.at[pl.ds(sid * rows_per_sub, rows_per_sub)],
                        acc_sh.at[pl.ds(sid * rows_per_sub, rows_per_sub)])
        plsc.subcore_barrier()

        def body(i, carry):
            base = wid * CHUNKS_PER_W + i * K
            pltpu.sync_copy(dst_hbm.at[pl.ds(base, K)], dst_v)
            for j in range(K):
                pltpu.sync_copy(ones_v, acc_sh.at[dst_v.at[j]], add=True)
            return carry

        lax.fori_loop(0, n_iter, body, 0)
        plsc.subcore_barrier()
        pltpu.sync_copy(acc_sh.at[pl.ds(sid * rows_per_sub, rows_per_sub)],
                        out_hbm.at[cid].at[pl.ds(sid * rows_per_sub, rows_per_sub)])

    return k(dst2d, zero_col, ones128)


_BR = 1024


def _tc_dinv(degpart):
    def body(d_ref, o_ref):
        d = d_ref[0] + d_ref[1] + 1.0
        o_ref[...] = lax.rsqrt(d)

    return pl.pallas_call(
        body,
        out_shape=jax.ShapeDtypeStruct((N_PAD, 1), jnp.float32),
    )(degpart)


def _tc_mm(x, w, b, scale=None, relu=False):
    Din, Dout = w.shape
    grid = (N_PAD // _BR,)
    in_specs = [
        pl.BlockSpec((_BR, Din), lambda i: (i, 0)),
        pl.BlockSpec((Din, Dout), lambda i: (0, 0)),
        pl.BlockSpec((1, Dout), lambda i: (0, 0)),
    ]
    args = [x, w, b.reshape(1, Dout)]
    if scale is not None:
        in_specs.append(pl.BlockSpec((_BR, 1), lambda i: (i, 0)))
        args.append(scale)

    def body(x_ref, w_ref, b_ref, *rest):
        o_ref = rest[-1]
        y = jnp.dot(x_ref[...], w_ref[...], preferred_element_type=jnp.float32)
        y = y + b_ref[...]
        if scale is not None:
            y = y * rest[0][...]
        if relu:
            y = jnp.maximum(y, 0.0)
        o_ref[...] = y

    return pl.pallas_call(
        body,
        grid=grid,
        in_specs=in_specs,
        out_specs=pl.BlockSpec((_BR, Dout), lambda i: (i, 0)),
        out_shape=jax.ShapeDtypeStruct((N_PAD, Dout), jnp.float32),
    )(*args)


def _tc_post(acc, h, dinv, b):
    D = h.shape[1]
    grid = (N_PAD // _BR,)

    def body(a_ref, h_ref, s_ref, b_ref, o_ref):
        t = a_ref[0] + a_ref[1] + h_ref[...]
        o_ref[...] = jnp.maximum(t * s_ref[...] + b_ref[...], 0.0)

    return pl.pallas_call(
        body,
        grid=grid,
        in_specs=[
            pl.BlockSpec((NC, _BR, D), lambda i: (0, i, 0)),
            pl.BlockSpec((_BR, D), lambda i: (i, 0)),
            pl.BlockSpec((_BR, 1), lambda i: (i, 0)),
            pl.BlockSpec((1, D), lambda i: (0, 0)),
        ],
        out_specs=pl.BlockSpec((_BR, D), lambda i: (i, 0)),
        out_shape=jax.ShapeDtypeStruct((N_PAD, D), jnp.float32),
    )(acc, h, dinv, b.reshape(1, D))


def _tc_attention(q, k, v):
    BQ = 512
    Dh = q.shape[1]
    scale = 1.0 / (Dh ** 0.5)
    grid = (N_PAD // BQ,)

    def body(q_ref, k_ref, v_ref, o_ref):
        s = lax.dot_general(q_ref[...], k_ref[...],
                            (((1,), (1,)), ((), ())),
                            preferred_element_type=jnp.float32) * scale
        col = lax.broadcasted_iota(jnp.int32, (BQ, N_PAD), 1)
        s = jnp.where(col < N, s, -1e30)
        m = jnp.max(s, axis=1, keepdims=True)
        p = jnp.exp(s - m)
        l = jnp.sum(p, axis=1, keepdims=True)
        o_ref[...] = jnp.dot(p, v_ref[...],
                             preferred_element_type=jnp.float32) / l

    return pl.pallas_call(
        body,
        grid=grid,
        in_specs=[
            pl.BlockSpec((BQ, Dh), lambda i: (i, 0)),
            pl.BlockSpec((N_PAD, Dh), lambda i: (0, 0)),
            pl.BlockSpec((N_PAD, Dh), lambda i: (0, 0)),
        ],
        out_specs=pl.BlockSpec((BQ, Dh), lambda i: (i, 0)),
        out_shape=jax.ShapeDtypeStruct((N_PAD, Dh), jnp.float32),
    )(q, k, v)


def _block_diag(a, b):
    r1, c1 = a.shape
    r2, c2 = b.shape
    top = jnp.concatenate([a, jnp.zeros((r1, c2), a.dtype)], axis=1)
    bot = jnp.concatenate([jnp.zeros((r2, c1), b.dtype), b], axis=1)
    return jnp.concatenate([top, bot], axis=0)


def kernel(adj, features, W1, b1, W2, b2, W3, b3, attn_in_w, attn_in_b,
           attn_out_w, attn_out_b, W4, b4, W5a, b5a, W6a, b6a, W7a, b7a,
           W5f, b5f, W6f, b6f, W7f, b7f):
    f32 = jnp.float32

    x0 = jnp.pad(features, ((0, N_PAD - N), (0, 0)))
    pad_idx = (N + (jnp.arange(E_PAD - E, dtype=jnp.int32) % (N_PAD - N))).astype(jnp.int32)
    src2d = jnp.concatenate([adj[0], pad_idx]).reshape(ROWS128, 128)
    dst2d = jnp.concatenate([adj[1], pad_idx]).reshape(ROWS128, 128)

    zero_cache = {}

    def zrows(D):
        if D not in zero_cache:
            zero_cache[D] = jnp.zeros((N_PAD, D), f32)
        return zero_cache[D]

    ones128 = jnp.ones((128, 1), f32)

    degp = _sc_degree(dst2d, zrows(1), ones128)
    dinv = _tc_dinv(degp)

    def conv(x, w, b, K):
        D = w.shape[1]
        h = _tc_mm(x, w, jnp.zeros((D,), f32), scale=dinv)
        if D <= 64:
            acc = _sc_edge_scatter(h, src2d, dst2d, zrows(D), D, K)
        else:
            dl = 64
            dr = D - dl
            acc_l = _sc_edge_scatter(h[:, :dl], src2d, dst2d, zrows(dl), dl, 8)
            acc_r = _sc_edge_scatter(h[:, dl:], src2d, dst2d, zrows(dr), dr, 8)
            acc = jnp.concatenate([acc_l, acc_r], axis=2)
        return h, acc

    h, acc = conv(x0, W1, b1, K=8)
    x = _tc_post(acc, h, dinv, b1)
    h, acc = conv(x, W2, b2, K=8)
    x = _tc_post(acc, h, dinv, b2)
    h, acc = conv(x, W3, b3, K=8)
    x = _tc_post(acc, h, dinv, b3)

    qkv = _tc_mm(x, attn_in_w.T, attn_in_b)
    q, kk, vv = qkv[:, :32], qkv[:, 32:64], qkv[:, 64:]
    a = _tc_attention(q, kk, vv)
    p = _tc_mm(a, attn_out_w.T, attn_out_b)

    h, acc = conv(p, W4, b4, K=8)
    x = _tc_post(acc, h, dinv, b4)

    W5 = jnp.concatenate([W5a, W5f], axis=1)
    b5 = jnp.concatenate([b5a, b5f])
    h, acc = conv(x, W5, b5, K=8)
    x = _tc_post(acc, h, dinv, b5)

    W6 = _block_diag(W6a, W6f)
    b6 = jnp.concatenate([b6a, b6f])
    h, acc = conv(x, W6, b6, K=4)
    x = _tc_post(acc, h, dinv, b6)

    W7 = _block_diag(W7a, W7f)
    b7 = jnp.concatenate([b7a, b7f])
    h, acc = conv(x, W7, b7, K=4)
    out = _tc_post(acc, h, dinv, b7)

    da = out[:N, :2]
    df = out[:N, 2:130]
    return (da, df)


# speedup vs baseline: 21.5142x; 1.2354x over previous
import functools

import jax
import jax.numpy as jnp
from jax import lax
from jax.experimental import pallas as pl
from jax.experimental.pallas import tpu as pltpu
from jax.experimental.pallas import tpu_sc as plsc

N = 10000
N_PAD = 10240
E = 320000
E_PAD = 327680
ROWS128 = E_PAD // 128
NC, NS = 2, 16
NW = NC * NS
CHUNKS_PER_W = ROWS128 // NW


def _sc_edge_scatter(h, src2d, dst2d, zero_rows, D, K=4):
    mesh = plsc.VectorSubcoreMesh(core_axis_name="c", subcore_axis_name="s")
    n_iter = CHUNKS_PER_W // K
    assert n_iter % 2 == 0 and n_iter >= 4
    rows_per_sub = N_PAD // NS

    @functools.partial(
        pl.kernel,
        out_type=jax.ShapeDtypeStruct((NC, N_PAD, D), jnp.float32),
        mesh=mesh,
        scratch_types=[
            pltpu.VMEM((CHUNKS_PER_W, 128), jnp.int32),
            pltpu.VMEM((K, 128), jnp.int32),
            pltpu.VMEM((K, 128), jnp.int32),
            pltpu.VMEM((K, 128, D), jnp.float32),
            pltpu.VMEM((K, 128, D), jnp.float32),
            pltpu.VMEM_SHARED((N_PAD, D), jnp.float32),
            pltpu.SemaphoreType.DMA,
            pltpu.SemaphoreType.DMA,
            pltpu.SemaphoreType.DMA,
            pltpu.SemaphoreType.DMA,
            pltpu.SemaphoreType.DMA,
            pltpu.SemaphoreType.DMA,
        ],
        compiler_params=pltpu.CompilerParams(use_tc_tiling_on_sc=False),
    )
    def k(h_hbm, src_hbm, dst_hbm, zero_hbm, out_hbm,
          src_all, dst0, dst1, rows0, rows1, acc_sh,
          gs0, gs1, ss0, ss1, is0, is1):
        cid = lax.axis_index("c")
        sid = lax.axis_index("s")
        wid = sid * NC + cid
        rows = (rows0, rows1)
        dstv = (dst0, dst1)
        gsem = (gs0, gs1)
        ssem = (ss0, ss1)
        isem = (is0, is1)

        pltpu.sync_copy(src_hbm.at[pl.ds(wid * CHUNKS_PER_W, CHUNKS_PER_W)], src_all)
        pltpu.sync_copy(zero_hbm.at[pl.ds(sid * rows_per_sub, rows_per_sub)],
                        acc_sh.at[pl.ds(sid * rows_per_sub, rows_per_sub)])
        plsc.subcore_barrier()

        def d_cp(ci, b):
            base = wid * CHUNKS_PER_W + ci * K
            return pltpu.make_async_copy(dst_hbm.at[pl.ds(base, K)], dstv[b], isem[b])

        def g_cp(ci, b, j):
            return pltpu.make_async_copy(
                h_hbm.at[src_all.at[ci * K + j]], rows[b].at[j], gsem[b])

        def s_cp(b, j):
            return pltpu.make_async_copy(
                rows[b].at[j], acc_sh.at[dstv[b].at[j]], ssem[b])

        def fire_dst(ci, b):
            d_cp(ci, b).start()

        def wait_dst(ci, b):
            d_cp(ci, b).wait()

        def fire_gather(ci, b):
            for j in range(K):
                g_cp(ci, b, j).start()

        def wait_gather(ci, b):
            for j in range(K):
                g_cp(ci, b, j).wait()

        def fire_scatter(b):
            for j in range(K):
                s_cp(b, j).start(add=True)

        def wait_scatter(b):
            for j in range(K):
                s_cp(b, j).wait()

        fire_dst(0, 0)
        fire_gather(0, 0)
        fire_dst(1, 1)
        fire_gather(1, 1)
        wait_gather(0, 0)
        wait_dst(0, 0)
        fire_scatter(0)

        def body(tt, carry):
            for b in (0, 1):
                i = 2 * tt + b
                wait_scatter(b)
                fire_dst(i, b)
                fire_gather(i, b)
                wait_gather(i - 1, 1 - b)
                wait_dst(i - 1, 1 - b)
                fire_scatter(1 - b)
            return carry

        lax.fori_loop(1, n_iter // 2, body, 0)

        last = n_iter - 1
        wait_gather(last, 1)
        wait_dst(last, 1)
        fire_scatter(1)
        wait_scatter(0)
        wait_scatter(1)

        plsc.subcore_barrier()
        pltpu.sync_copy(acc_sh.at[pl.ds(sid * rows_per_sub, rows_per_sub)],
                        out_hbm.at[cid].at[pl.ds(sid * rows_per_sub, rows_per_sub)])

    return k(h, src2d, dst2d, zero_rows)


def _sc_degree(dst2d, zero_col, ones128):
    mesh = plsc.VectorSubcoreMesh(core_axis_name="c", subcore_axis_name="s")
    K = 8
    n_iter = CHUNKS_PER_W // K
    rows_per_sub = N_PAD // NS

    @functools.partial(
        pl.kernel,
        out_type=jax.ShapeDtypeStruct((NC, N_PAD, 16), jnp.float32),
        mesh=mesh,
        scratch_types=[
            pltpu.VMEM((K, 128), jnp.int32),
            pltpu.VMEM((128, 16), jnp.float32),
            pltpu.VMEM_SHARED((N_PAD, 16), jnp.float32),
        ],
        compiler_params=pltpu.CompilerParams(use_tc_tiling_on_sc=False),
    )
    def k(dst_hbm, zero_hbm, ones_hbm, out_hbm, dst_v, ones_v, acc_sh):
        cid = lax.axis_index("c")
        sid = lax.axis_index("s")
        wid = sid * NC + cid
        pltpu.sync_copy(ones_hbm, ones_v)
        pltpu.sync_copy(zero_hbm.at[pl.ds(sid * rows_per_sub, rows_per_sub)],
                        acc_sh.at[pl.ds(sid * rows_per_sub, rows_per_sub)])
        plsc.subcore_barrier()

        def body(i, carry):
            base = wid * CHUNKS_PER_W + i * K
            pltpu.sync_copy(dst_hbm.at[pl.ds(base, K)], dst_v)
            for j in range(K):
                pltpu.sync_copy(ones_v, acc_sh.at[dst_v.at[j]], add=True)
            return carry

        lax.fori_loop(0, n_iter, body, 0)
        plsc.subcore_barrier()
        pltpu.sync_copy(acc_sh.at[pl.ds(sid * rows_per_sub, rows_per_sub)],
                        out_hbm.at[cid].at[pl.ds(sid * rows_per_sub, rows_per_sub)])

    return k(dst2d, zero_col, ones128)


_BR = 1024


def _tc_dinv(degpart):
    def body(d_ref, o_ref):
        d = d_ref[0] + d_ref[1] + 1.0
        o_ref[...] = lax.rsqrt(d)

    return pl.pallas_call(
        body,
        out_shape=jax.ShapeDtypeStruct((N_PAD, 1), jnp.float32),
    )(degpart)


def _tc_mm(x, w, b, scale=None, relu=False):
    Din, Dout = w.shape
    grid = (N_PAD // _BR,)
    in_specs = [
        pl.BlockSpec((_BR, Din), lambda i: (i, 0)),
        pl.BlockSpec((Din, Dout), lambda i: (0, 0)),
        pl.BlockSpec((1, Dout), lambda i: (0, 0)),
    ]
    args = [x, w, b.reshape(1, Dout)]
    if scale is not None:
        in_specs.append(pl.BlockSpec((_BR, 1), lambda i: (i, 0)))
        args.append(scale)

    def body(x_ref, w_ref, b_ref, *rest):
        o_ref = rest[-1]
        y = jnp.dot(x_ref[...], w_ref[...], preferred_element_type=jnp.float32)
        y = y + b_ref[...]
        if scale is not None:
            y = y * rest[0][...]
        if relu:
            y = jnp.maximum(y, 0.0)
        o_ref[...] = y

    return pl.pallas_call(
        body,
        grid=grid,
        in_specs=in_specs,
        out_specs=pl.BlockSpec((_BR, Dout), lambda i: (i, 0)),
        out_shape=jax.ShapeDtypeStruct((N_PAD, Dout), jnp.float32),
    )(*args)


def _tc_post(acc, h, dinv, b):
    D = h.shape[1]
    grid = (N_PAD // _BR,)

    def body(a_ref, h_ref, s_ref, b_ref, o_ref):
        t = a_ref[0] + a_ref[1] + h_ref[...]
        o_ref[...] = jnp.maximum(t * s_ref[...] + b_ref[...], 0.0)

    return pl.pallas_call(
        body,
        grid=grid,
        in_specs=[
            pl.BlockSpec((NC, _BR, D), lambda i: (0, i, 0)),
            pl.BlockSpec((_BR, D), lambda i: (i, 0)),
            pl.BlockSpec((_BR, 1), lambda i: (i, 0)),
            pl.BlockSpec((1, D), lambda i: (0, 0)),
        ],
        out_specs=pl.BlockSpec((_BR, D), lambda i: (i, 0)),
        out_shape=jax.ShapeDtypeStruct((N_PAD, D), jnp.float32),
    )(acc, h, dinv, b.reshape(1, D))


def _tc_attention(q, k, v):
    BQ = 512
    Dh = q.shape[1]
    scale = 1.0 / (Dh ** 0.5)
    grid = (N_PAD // BQ,)

    def body(q_ref, k_ref, v_ref, o_ref):
        s = lax.dot_general(q_ref[...], k_ref[...],
                            (((1,), (1,)), ((), ())),
                            preferred_element_type=jnp.float32) * scale
        col = lax.broadcasted_iota(jnp.int32, (BQ, N_PAD), 1)
        s = jnp.where(col < N, s, -1e30)
        m = jnp.max(s, axis=1, keepdims=True)
        p = jnp.exp(s - m)
        l = jnp.sum(p, axis=1, keepdims=True)
        o_ref[...] = jnp.dot(p, v_ref[...],
                             preferred_element_type=jnp.float32) / l

    return pl.pallas_call(
        body,
        grid=grid,
        in_specs=[
            pl.BlockSpec((BQ, Dh), lambda i: (i, 0)),
            pl.BlockSpec((N_PAD, Dh), lambda i: (0, 0)),
            pl.BlockSpec((N_PAD, Dh), lambda i: (0, 0)),
        ],
        out_specs=pl.BlockSpec((BQ, Dh), lambda i: (i, 0)),
        out_shape=jax.ShapeDtypeStruct((N_PAD, Dh), jnp.float32),
    )(q, k, v)


def _block_diag(a, b):
    r1, c1 = a.shape
    r2, c2 = b.shape
    top = jnp.concatenate([a, jnp.zeros((r1, c2), a.dtype)], axis=1)
    bot = jnp.concatenate([jnp.zeros((r2, c1), b.dtype), b], axis=1)
    return jnp.concatenate([top, bot], axis=0)


def kernel(adj, features, W1, b1, W2, b2, W3, b3, attn_in_w, attn_in_b,
           attn_out_w, attn_out_b, W4, b4, W5a, b5a, W6a, b6a, W7a, b7a,
           W5f, b5f, W6f, b6f, W7f, b7f):
    f32 = jnp.float32

    x0 = jnp.pad(features, ((0, N_PAD - N), (0, 0)))
    pad_idx = (N + (jnp.arange(E_PAD - E, dtype=jnp.int32) % (N_PAD - N))).astype(jnp.int32)
    src2d = jnp.concatenate([adj[0], pad_idx]).reshape(ROWS128, 128)
    dst2d = jnp.concatenate([adj[1], pad_idx]).reshape(ROWS128, 128)

    zero_cache = {}

    def zrows(D):
        if D not in zero_cache:
            zero_cache[D] = jnp.zeros((N_PAD, D), f32)
        return zero_cache[D]

    ones128 = jnp.ones((128, 16), f32)

    degp = _sc_degree(dst2d, zrows(16), ones128)
    dinv = _tc_dinv(degp[:, :, :1])

    def conv(x, w, b, K):
        D = w.shape[1]
        h = _tc_mm(x, w, jnp.zeros((D,), f32), scale=dinv)
        if D <= 64:
            acc = _sc_edge_scatter(h, src2d, dst2d, zrows(D), D, 4)
        else:
            dl = 64
            dr = D - dl
            acc_l = _sc_edge_scatter(h[:, :dl], src2d, dst2d, zrows(dl), dl, 4)
            acc_r = _sc_edge_scatter(h[:, dl:], src2d, dst2d, zrows(dr), dr,
                                     4 if dr <= 64 else 2)
            acc = jnp.concatenate([acc_l, acc_r], axis=2)
        return h, acc

    h, acc = conv(x0, W1, b1, K=4)
    x = _tc_post(acc, h, dinv, b1)
    h, acc = conv(x, W2, b2, K=4)
    x = _tc_post(acc, h, dinv, b2)
    h, acc = conv(x, W3, b3, K=4)
    x = _tc_post(acc, h, dinv, b3)

    qkv = _tc_mm(x, attn_in_w.T, attn_in_b)
    q, kk, vv = qkv[:, :32], qkv[:, 32:64], qkv[:, 64:]
    a = _tc_attention(q, kk, vv)
    p = _tc_mm(a, attn_out_w.T, attn_out_b)

    h, acc = conv(p, W4, b4, K=4)
    x = _tc_post(acc, h, dinv, b4)

    W5 = jnp.concatenate([W5a, W5f], axis=1)
    b5 = jnp.concatenate([b5a, b5f])
    h, acc = conv(x, W5, b5, K=4)
    x = _tc_post(acc, h, dinv, b5)

    W6 = _block_diag(W6a, W6f)
    b6 = jnp.concatenate([b6a, b6f])
    h, acc = conv(x, W6, b6, K=4)
    x = _tc_post(acc, h, dinv, b6)

    W7 = jnp.pad(_block_diag(W7a, W7f), ((0, 0), (0, 14)))
    b7 = jnp.pad(jnp.concatenate([b7a, b7f]), (0, 14))
    h, acc = conv(x, W7, b7, K=4)
    out = _tc_post(acc, h, dinv, b7)

    da = out[:N, :2]
    df = out[:N, 2:130]
    return (da, df)


# speedup vs baseline: 23.7132x; 1.1022x over previous
import functools

import jax
import jax.numpy as jnp
from jax import lax
from jax.experimental import pallas as pl
from jax.experimental.pallas import tpu as pltpu
from jax.experimental.pallas import tpu_sc as plsc

N = 10000
N_PAD = 10240
E = 320000
E_PAD = 327680
ROWS128 = E_PAD // 128
NC, NS = 2, 16
NW = NC * NS
CHUNKS_PER_W = ROWS128 // NW
CHUNKS_PER_S = ROWS128 // NS


def _sc_edge_scatter(h, src2d, dst2d, zero_rows, D, K=4):
    mesh = plsc.VectorSubcoreMesh(core_axis_name="c", subcore_axis_name="s")
    n_iter = CHUNKS_PER_W // K
    assert n_iter % 2 == 0 and n_iter >= 4
    rows_per_sub = N_PAD // NS

    @functools.partial(
        pl.kernel,
        out_type=jax.ShapeDtypeStruct((NC, N_PAD, D), jnp.float32),
        mesh=mesh,
        scratch_types=[
            pltpu.VMEM((CHUNKS_PER_W, 128), jnp.int32),
            pltpu.VMEM((K, 128), jnp.int32),
            pltpu.VMEM((K, 128), jnp.int32),
            pltpu.VMEM((K, 128, D), jnp.float32),
            pltpu.VMEM((K, 128, D), jnp.float32),
            pltpu.VMEM_SHARED((N_PAD, D), jnp.float32),
            pltpu.SemaphoreType.DMA,
            pltpu.SemaphoreType.DMA,
            pltpu.SemaphoreType.DMA,
            pltpu.SemaphoreType.DMA,
            pltpu.SemaphoreType.DMA,
            pltpu.SemaphoreType.DMA,
        ],
        compiler_params=pltpu.CompilerParams(use_tc_tiling_on_sc=False),
    )
    def k(h_hbm, src_hbm, dst_hbm, zero_hbm, out_hbm,
          src_all, dst0, dst1, rows0, rows1, acc_sh,
          gs0, gs1, ss0, ss1, is0, is1):
        cid = lax.axis_index("c")
        sid = lax.axis_index("s")
        wid = sid * NC + cid
        rows = (rows0, rows1)
        dstv = (dst0, dst1)
        gsem = (gs0, gs1)
        ssem = (ss0, ss1)
        isem = (is0, is1)

        pltpu.sync_copy(src_hbm.at[pl.ds(wid * CHUNKS_PER_W, CHUNKS_PER_W)], src_all)
        pltpu.sync_copy(zero_hbm.at[pl.ds(sid * rows_per_sub, rows_per_sub)],
                        acc_sh.at[pl.ds(sid * rows_per_sub, rows_per_sub)])
        plsc.subcore_barrier()

        def d_cp(ci, b):
            base = wid * CHUNKS_PER_W + ci * K
            return pltpu.make_async_copy(dst_hbm.at[pl.ds(base, K)], dstv[b], isem[b])

        def g_cp(ci, b, j):
            return pltpu.make_async_copy(
                h_hbm.at[src_all.at[ci * K + j]], rows[b].at[j], gsem[b])

        def s_cp(b, j):
            return pltpu.make_async_copy(
                rows[b].at[j], acc_sh.at[dstv[b].at[j]], ssem[b])

        def fire_gather(ci, b):
            for j in range(K):
                g_cp(ci, b, j).start()

        def wait_gather(ci, b):
            for j in range(K):
                g_cp(ci, b, j).wait()

        def fire_scatter(b):
            for j in range(K):
                s_cp(b, j).start(add=True)

        def wait_scatter(b):
            for j in range(K):
                s_cp(b, j).wait()

        d_cp(0, 0).start()
        fire_gather(0, 0)
        d_cp(1, 1).start()
        fire_gather(1, 1)
        wait_gather(0, 0)
        d_cp(0, 0).wait()
        fire_scatter(0)

        def body(tt, carry):
            for b in (0, 1):
                i = 2 * tt + b
                wait_scatter(b)
                d_cp(i, b).start()
                fire_gather(i, b)
                wait_gather(i - 1, 1 - b)
                d_cp(i - 1, 1 - b).wait()
                fire_scatter(1 - b)
            return carry

        lax.fori_loop(1, n_iter // 2, body, 0)

        last = n_iter - 1
        wait_gather(last, 1)
        d_cp(last, 1).wait()
        fire_scatter(1)
        wait_scatter(0)
        wait_scatter(1)

        plsc.subcore_barrier()
        pltpu.sync_copy(acc_sh.at[pl.ds(sid * rows_per_sub, rows_per_sub)],
                        out_hbm.at[cid].at[pl.ds(sid * rows_per_sub, rows_per_sub)])

    return k(h, src2d, dst2d, zero_rows)


def _sc_edge_scatter_pair(h2, src2d, dst2d, zero_rows, D, K=2):
    mesh = plsc.VectorSubcoreMesh(core_axis_name="c", subcore_axis_name="s")
    n_iter = CHUNKS_PER_S // K
    assert n_iter % 2 == 0 and n_iter >= 4
    rows_per_sub = N_PAD // NS

    @functools.partial(
        pl.kernel,
        out_type=jax.ShapeDtypeStruct((NC, N_PAD, D), jnp.float32),
        mesh=mesh,
        scratch_types=[
            pltpu.VMEM((CHUNKS_PER_S, 128), jnp.int32),
            pltpu.VMEM((K, 128), jnp.int32),
            pltpu.VMEM((K, 128), jnp.int32),
            pltpu.VMEM((K, 128, D), jnp.float32),
            pltpu.VMEM((K, 128, D), jnp.float32),
            pltpu.VMEM_SHARED((N_PAD, D), jnp.float32),
            pltpu.SemaphoreType.DMA,
            pltpu.SemaphoreType.DMA,
            pltpu.SemaphoreType.DMA,
            pltpu.SemaphoreType.DMA,
            pltpu.SemaphoreType.DMA,
            pltpu.SemaphoreType.DMA,
        ],
        compiler_params=pltpu.CompilerParams(use_tc_tiling_on_sc=False),
    )
    def k(hl_hbm, hr_hbm, src_hbm, dst_hbm, zero_hbm, out_hbm,
          src_all, dst0, dst1, rows0, rows1, acc_sh,
          gs0, gs1, ss0, ss1, is0, is1):
        cid = lax.axis_index("c")
        sid = lax.axis_index("s")
        rows = (rows0, rows1)
        dstv = (dst0, dst1)
        gsem = (gs0, gs1)
        ssem = (ss0, ss1)
        isem = (is0, is1)

        pltpu.sync_copy(src_hbm.at[pl.ds(sid * CHUNKS_PER_S, CHUNKS_PER_S)], src_all)
        pltpu.sync_copy(zero_hbm.at[pl.ds(sid * rows_per_sub, rows_per_sub)],
                        acc_sh.at[pl.ds(sid * rows_per_sub, rows_per_sub)])
        plsc.subcore_barrier()

        def d_cp(ci, b):
            base = sid * CHUNKS_PER_S + ci * K
            return pltpu.make_async_copy(dst_hbm.at[pl.ds(base, K)], dstv[b], isem[b])

        def s_cp(b, j):
            return pltpu.make_async_copy(
                rows[b].at[j], acc_sh.at[dstv[b].at[j]], ssem[b])

        def fire_scatter(b):
            for j in range(K):
                s_cp(b, j).start(add=True)

        def wait_scatter(b):
            for j in range(K):
                s_cp(b, j).wait()

        def run(h_hbm):
            def g_cp(ci, b, j):
                return pltpu.make_async_copy(
                    h_hbm.at[src_all.at[ci * K + j]], rows[b].at[j], gsem[b])

            def fire_gather(ci, b):
                for j in range(K):
                    g_cp(ci, b, j).start()

            def wait_gather(ci, b):
                for j in range(K):
                    g_cp(ci, b, j).wait()

            d_cp(0, 0).start()
            fire_gather(0, 0)
            d_cp(1, 1).start()
            fire_gather(1, 1)
            wait_gather(0, 0)
            d_cp(0, 0).wait()
            fire_scatter(0)

            def body(tt, carry):
                for b in (0, 1):
                    i = 2 * tt + b
                    wait_scatter(b)
                    d_cp(i, b).start()
                    fire_gather(i, b)
                    wait_gather(i - 1, 1 - b)
                    d_cp(i - 1, 1 - b).wait()
                    fire_scatter(1 - b)
                return carry

            lax.fori_loop(1, n_iter // 2, body, 0)

            last = n_iter - 1
            wait_gather(last, 1)
            d_cp(last, 1).wait()
            fire_scatter(1)
            wait_scatter(0)
            wait_scatter(1)

        @pl.when(cid == 0)
        def _():
            run(hl_hbm)

        @pl.when(cid == 1)
        def _():
            run(hr_hbm)

        plsc.subcore_barrier()
        pltpu.sync_copy(acc_sh.at[pl.ds(sid * rows_per_sub, rows_per_sub)],
                        out_hbm.at[cid].at[pl.ds(sid * rows_per_sub, rows_per_sub)])

    return k(h2[0], h2[1], src2d, dst2d, zero_rows)


def _sc_degree(dst2d, zero_col, ones128):
    mesh = plsc.VectorSubcoreMesh(core_axis_name="c", subcore_axis_name="s")
    K = 8
    n_iter = CHUNKS_PER_W // K
    rows_per_sub = N_PAD // NS

    @functools.partial(
        pl.kernel,
        out_type=jax.ShapeDtypeStruct((NC, N_PAD, 16), jnp.float32),
        mesh=mesh,
        scratch_types=[
            pltpu.VMEM((K, 128), jnp.int32),
            pltpu.VMEM((K, 128), jnp.int32),
            pltpu.VMEM((128, 16), jnp.float32),
            pltpu.VMEM_SHARED((N_PAD, 16), jnp.float32),
            pltpu.SemaphoreType.DMA,
            pltpu.SemaphoreType.DMA,
            pltpu.SemaphoreType.DMA,
            pltpu.SemaphoreType.DMA,
        ],
        compiler_params=pltpu.CompilerParams(use_tc_tiling_on_sc=False),
    )
    def k(dst_hbm, zero_hbm, ones_hbm, out_hbm, dst0, dst1, ones_v, acc_sh,
          ss0, ss1, is0, is1):
        cid = lax.axis_index("c")
        sid = lax.axis_index("s")
        wid = sid * NC + cid
        dstv = (dst0, dst1)
        ssem = (ss0, ss1)
        isem = (is0, is1)
        pltpu.sync_copy(ones_hbm, ones_v)
        pltpu.sync_copy(zero_hbm.at[pl.ds(sid * rows_per_sub, rows_per_sub)],
                        acc_sh.at[pl.ds(sid * rows_per_sub, rows_per_sub)])
        plsc.subcore_barrier()

        def d_cp(ci, b):
            base = wid * CHUNKS_PER_W + ci * K
            return pltpu.make_async_copy(dst_hbm.at[pl.ds(base, K)], dstv[b], isem[b])

        def s_cp(b, j):
            return pltpu.make_async_copy(ones_v, acc_sh.at[dstv[b].at[j]], ssem[b])

        def fire_scatter(b):
            for j in range(K):
                s_cp(b, j).start(add=True)

        def wait_scatter(b):
            for j in range(K):
                s_cp(b, j).wait()

        d_cp(0, 0).start()
        d_cp(1, 1).start()
        d_cp(0, 0).wait()
        fire_scatter(0)

        def body(tt, carry):
            for b in (0, 1):
                i = 2 * tt + b
                wait_scatter(b)
                d_cp(i, b).start()
                d_cp(i - 1, 1 - b).wait()
                fire_scatter(1 - b)
            return carry

        lax.fori_loop(1, n_iter // 2, body, 0)

        last = n_iter - 1
        d_cp(last, 1).wait()
        fire_scatter(1)
        wait_scatter(0)
        wait_scatter(1)

        plsc.subcore_barrier()
        pltpu.sync_copy(acc_sh.at[pl.ds(sid * rows_per_sub, rows_per_sub)],
                        out_hbm.at[cid].at[pl.ds(sid * rows_per_sub, rows_per_sub)])

    return k(dst2d, zero_col, ones128)


_BR = 1024


def _tc_dinv(degpart):
    def body(d_ref, o_ref):
        d = d_ref[0] + d_ref[1] + 1.0
        o_ref[...] = lax.rsqrt(d)

    return pl.pallas_call(
        body,
        out_shape=jax.ShapeDtypeStruct((N_PAD, 1), jnp.float32),
    )(degpart)


def _tc_mm(x, w, b, scale=None, relu=False):
    Din, Dout = w.shape
    grid = (N_PAD // _BR,)
    in_specs = [
        pl.BlockSpec((_BR, Din), lambda i: (i, 0)),
        pl.BlockSpec((Din, Dout), lambda i: (0, 0)),
        pl.BlockSpec((1, Dout), lambda i: (0, 0)),
    ]
    args = [x, w, b.reshape(1, Dout)]
    if scale is not None:
        in_specs.append(pl.BlockSpec((_BR, 1), lambda i: (i, 0)))
        args.append(scale)

    def body(x_ref, w_ref, b_ref, *rest):
        o_ref = rest[-1]
        y = jnp.dot(x_ref[...], w_ref[...], preferred_element_type=jnp.float32)
        y = y + b_ref[...]
        if scale is not None:
            y = y * rest[0][...]
        if relu:
            y = jnp.maximum(y, 0.0)
        o_ref[...] = y

    return pl.pallas_call(
        body,
        grid=grid,
        in_specs=in_specs,
        out_specs=pl.BlockSpec((_BR, Dout), lambda i: (i, 0)),
        out_shape=jax.ShapeDtypeStruct((N_PAD, Dout), jnp.float32),
    )(*args)


def _tc_stage(acc, h, dinv, b, w, b2, scale_out=True, in_halves=False,
              out_split=0):
    Dx = acc.shape[2] * (2 if in_halves else 1)
    Din, Dout = w.shape
    assert Din == Dx
    grid = (N_PAD // _BR,)
    Dh = acc.shape[2]

    def body(a_ref, h_ref, s_ref, b_ref, w_ref, b2_ref, o_ref):
        if in_halves:
            t = (jnp.concatenate([a_ref[0], a_ref[1]], axis=1)
                 + jnp.concatenate([h_ref[0], h_ref[1]], axis=1))
        else:
            t = a_ref[0] + a_ref[1] + h_ref[...]
        s = s_ref[...]
        x = jnp.maximum(t * s + b_ref[...], 0.0)
        y = jnp.dot(x, w_ref[...], preferred_element_type=jnp.float32)
        y = y + b2_ref[...]
        if scale_out:
            y = y * s
        if out_split:
            o_ref[0] = y[:, :out_split]
            o_ref[1] = y[:, out_split:]
        else:
            o_ref[...] = y

    h_spec = (pl.BlockSpec((2, _BR, Dh), lambda i: (0, i, 0)) if in_halves
              else pl.BlockSpec((_BR, Dx), lambda i: (i, 0)))
    if out_split:
        out_spec = pl.BlockSpec((2, _BR, out_split), lambda i: (0, i, 0))
        out_shape = jax.ShapeDtypeStruct((2, N_PAD, out_split), jnp.float32)
    else:
        out_spec = pl.BlockSpec((_BR, Dout), lambda i: (i, 0))
        out_shape = jax.ShapeDtypeStruct((N_PAD, Dout), jnp.float32)

    return pl.pallas_call(
        body,
        grid=grid,
        in_specs=[
            pl.BlockSpec((2, _BR, Dh), lambda i: (0, i, 0)),
            h_spec,
            pl.BlockSpec((_BR, 1), lambda i: (i, 0)),
            pl.BlockSpec((1, Dx), lambda i: (0, 0)),
            pl.BlockSpec((Din, Dout), lambda i: (0, 0)),
            pl.BlockSpec((1, Dout), lambda i: (0, 0)),
        ],
        out_specs=out_spec,
        out_shape=out_shape,
    )(acc, h, dinv, b.reshape(1, Dx), w, b2.reshape(1, Dout))


def _tc_post(acc, h, dinv, b, in_halves=False):
    Dh = acc.shape[2]
    D = Dh * (2 if in_halves else 1)
    grid = (N_PAD // _BR,)

    def body(a_ref, h_ref, s_ref, b_ref, o_ref):
        if in_halves:
            t = (jnp.concatenate([a_ref[0], a_ref[1]], axis=1)
                 + jnp.concatenate([h_ref[0], h_ref[1]], axis=1))
        else:
            t = a_ref[0] + a_ref[1] + h_ref[...]
        o_ref[...] = jnp.maximum(t * s_ref[...] + b_ref[...], 0.0)

    h_spec = (pl.BlockSpec((2, _BR, Dh), lambda i: (0, i, 0)) if in_halves
              else pl.BlockSpec((_BR, D), lambda i: (i, 0)))
    return pl.pallas_call(
        body,
        grid=grid,
        in_specs=[
            pl.BlockSpec((2, _BR, Dh), lambda i: (0, i, 0)),
            h_spec,
            pl.BlockSpec((_BR, 1), lambda i: (i, 0)),
            pl.BlockSpec((1, D), lambda i: (0, 0)),
        ],
        out_specs=pl.BlockSpec((_BR, D), lambda i: (i, 0)),
        out_shape=jax.ShapeDtypeStruct((N_PAD, D), jnp.float32),
    )(acc, h, dinv, b.reshape(1, D))


def _tc_projproj(a, w1, b1, w2, dinv):
    D1 = w1.shape[1]
    D2 = w2.shape[1]
    grid = (N_PAD // _BR,)

    def body(a_ref, w1_ref, b1_ref, w2_ref, s_ref, o_ref):
        p = jnp.dot(a_ref[...], w1_ref[...], preferred_element_type=jnp.float32)
        p = p + b1_ref[...]
        y = jnp.dot(p, w2_ref[...], preferred_element_type=jnp.float32)
        o_ref[...] = y * s_ref[...]

    return pl.pallas_call(
        body,
        grid=grid,
        in_specs=[
            pl.BlockSpec((_BR, a.shape[1]), lambda i: (i, 0)),
            pl.BlockSpec(w1.shape, lambda i: (0, 0)),
            pl.BlockSpec((1, D1), lambda i: (0, 0)),
            pl.BlockSpec(w2.shape, lambda i: (0, 0)),
            pl.BlockSpec((_BR, 1), lambda i: (i, 0)),
        ],
        out_specs=pl.BlockSpec((_BR, D2), lambda i: (i, 0)),
        out_shape=jax.ShapeDtypeStruct((N_PAD, D2), jnp.float32),
    )(a, w1, b1.reshape(1, D1), w2, dinv)


def _tc_attention(q, k, v):
    BQ = 512
    Dh = q.shape[1]
    scale = 1.0 / (Dh ** 0.5)
    grid = (N_PAD // BQ,)

    def body(q_ref, k_ref, v_ref, o_ref):
        s = lax.dot_general(q_ref[...], k_ref[...],
                            (((1,), (1,)), ((), ())),
                            preferred_element_type=jnp.float32) * scale
        col = lax.broadcasted_iota(jnp.int32, (BQ, N_PAD), 1)
        s = jnp.where(col < N, s, -1e30)
        m = jnp.max(s, axis=1, keepdims=True)
        p = jnp.exp(s - m)
        l = jnp.sum(p, axis=1, keepdims=True)
        o_ref[...] = jnp.dot(p, v_ref[...],
                             preferred_element_type=jnp.float32) / l

    return pl.pallas_call(
        body,
        grid=grid,
        in_specs=[
            pl.BlockSpec((BQ, Dh), lambda i: (i, 0)),
            pl.BlockSpec((N_PAD, Dh), lambda i: (0, 0)),
            pl.BlockSpec((N_PAD, Dh), lambda i: (0, 0)),
        ],
        out_specs=pl.BlockSpec((BQ, Dh), lambda i: (i, 0)),
        out_shape=jax.ShapeDtypeStruct((N_PAD, Dh), jnp.float32),
    )(q, k, v)


def _block_diag(a, b):
    r1, c1 = a.shape
    r2, c2 = b.shape
    top = jnp.concatenate([a, jnp.zeros((r1, c2), a.dtype)], axis=1)
    bot = jnp.concatenate([jnp.zeros((r2, c1), b.dtype), b], axis=1)
    return jnp.concatenate([top, bot], axis=0)


def kernel(adj, features, W1, b1, W2, b2, W3, b3, attn_in_w, attn_in_b,
           attn_out_w, attn_out_b, W4, b4, W5a, b5a, W6a, b6a, W7a, b7a,
           W5f, b5f, W6f, b6f, W7f, b7f):
    f32 = jnp.float32

    x0 = jnp.pad(features, ((0, N_PAD - N), (0, 0)))
    pad_idx = (N + (jnp.arange(E_PAD - E, dtype=jnp.int32) % (N_PAD - N))).astype(jnp.int32)
    src2d = jnp.concatenate([adj[0], pad_idx]).reshape(ROWS128, 128)
    dst2d = jnp.concatenate([adj[1], pad_idx]).reshape(ROWS128, 128)

    zero_cache = {}

    def zrows(D):
        if D not in zero_cache:
            zero_cache[D] = jnp.zeros((N_PAD, D), f32)
        return zero_cache[D]

    ones128 = jnp.ones((128, 16), f32)
    z32 = jnp.zeros((32,), f32)
    z64 = jnp.zeros((64,), f32)

    degp = _sc_degree(dst2d, zrows(16), ones128)
    dinv = _tc_dinv(degp[:, :, :1])

    h = _tc_mm(x0, W1, z64, scale=dinv)
    acc = _sc_edge_scatter(h, src2d, dst2d, zrows(64), 64)
    h = _tc_stage(acc, h, dinv, b1, W2, z32)
    acc = _sc_edge_scatter(h, src2d, dst2d, zrows(32), 32)
    h = _tc_stage(acc, h, dinv, b2, W3, z32)
    acc = _sc_edge_scatter(h, src2d, dst2d, zrows(32), 32)
    qkv = _tc_stage(acc, h, dinv, b3, attn_in_w.T, attn_in_b, scale_out=False)

    a = _tc_attention(qkv[:, :32], qkv[:, 32:64], qkv[:, 64:])
    h = _tc_projproj(a, attn_out_w.T, attn_out_b, W4, dinv)
    acc = _sc_edge_scatter(h, src2d, dst2d, zrows(32), 32)

    W5 = jnp.concatenate([W5a, W5f], axis=1)
    b5 = jnp.concatenate([b5a, b5f])
    h = _tc_stage(acc, h, dinv, b4, W5, z64)
    acc = _sc_edge_scatter(h, src2d, dst2d, zrows(64), 64)

    W6 = _block_diag(W6a, W6f)
    b6 = jnp.concatenate([b6a, b6f])
    h = _tc_stage(acc, h, dinv, b5, W6, jnp.zeros((128,), f32), out_split=64)
    acc = _sc_edge_scatter_pair(h, src2d, dst2d, zrows(64), 64)

    W7 = jnp.pad(_block_diag(W7a, W7f), ((0, 0), (0, 30)))
    b7 = jnp.pad(jnp.concatenate([b7a, b7f]), (0, 30))
    h = _tc_stage(acc, h, dinv, b6, W7, jnp.zeros((160,), f32),
                  in_halves=True, out_split=80)
    acc = _sc_edge_scatter_pair(h, src2d, dst2d, zrows(80), 80)

    out = _tc_post(acc, h, dinv, b7, in_halves=True)
    da = out[:N, :2]
    df = out[:N, 2:130]
    return (da, df)


# speedup vs baseline: 23.7717x; 1.0025x over previous
import functools

import jax
import jax.numpy as jnp
from jax import lax
from jax.experimental import pallas as pl
from jax.experimental.pallas import tpu as pltpu
from jax.experimental.pallas import tpu_sc as plsc

N = 10000
N_PAD = 10240
E = 320000
E_PAD = 327680
ROWS128 = E_PAD // 128
NC, NS = 2, 16
NW = NC * NS
CHUNKS_PER_W = ROWS128 // NW
CHUNKS_PER_S = ROWS128 // NS


def _sc_edge_scatter(h, src2d, dst2d, zero_rows, D, K=4):
    mesh = plsc.VectorSubcoreMesh(core_axis_name="c", subcore_axis_name="s")
    n_iter = CHUNKS_PER_W // K
    assert n_iter % 2 == 0 and n_iter >= 4
    rows_per_sub = N_PAD // NS

    @functools.partial(
        pl.kernel,
        out_type=jax.ShapeDtypeStruct((NC, N_PAD, D), jnp.float32),
        mesh=mesh,
        scratch_types=[
            pltpu.VMEM((CHUNKS_PER_W, 128), jnp.int32),
            pltpu.VMEM((K, 128), jnp.int32),
            pltpu.VMEM((K, 128), jnp.int32),
            pltpu.VMEM((K, 128, D), jnp.float32),
            pltpu.VMEM((K, 128, D), jnp.float32),
            pltpu.VMEM_SHARED((N_PAD, D), jnp.float32),
            pltpu.SemaphoreType.DMA,
            pltpu.SemaphoreType.DMA,
            pltpu.SemaphoreType.DMA,
            pltpu.SemaphoreType.DMA,
            pltpu.SemaphoreType.DMA,
            pltpu.SemaphoreType.DMA,
        ],
        compiler_params=pltpu.CompilerParams(use_tc_tiling_on_sc=False),
    )
    def k(h_hbm, src_hbm, dst_hbm, zero_hbm, out_hbm,
          src_all, dst0, dst1, rows0, rows1, acc_sh,
          gs0, gs1, ss0, ss1, is0, is1):
        cid = lax.axis_index("c")
        sid = lax.axis_index("s")
        wid = sid * NC + cid
        rows = (rows0, rows1)
        dstv = (dst0, dst1)
        gsem = (gs0, gs1)
        ssem = (ss0, ss1)
        isem = (is0, is1)

        pltpu.sync_copy(src_hbm.at[pl.ds(wid * CHUNKS_PER_W, CHUNKS_PER_W)], src_all)
        pltpu.sync_copy(zero_hbm.at[pl.ds(sid * rows_per_sub, rows_per_sub)],
                        acc_sh.at[pl.ds(sid * rows_per_sub, rows_per_sub)])
        plsc.subcore_barrier()

        def d_cp(ci, b):
            base = wid * CHUNKS_PER_W + ci * K
            return pltpu.make_async_copy(dst_hbm.at[pl.ds(base, K)], dstv[b], isem[b])

        def g_cp(ci, b, j):
            return pltpu.make_async_copy(
                h_hbm.at[src_all.at[ci * K + j]], rows[b].at[j], gsem[b])

        def s_cp(b, j):
            return pltpu.make_async_copy(
                rows[b].at[j], acc_sh.at[dstv[b].at[j]], ssem[b])

        def fire_gather(ci, b):
            for j in range(K):
                g_cp(ci, b, j).start()

        def wait_gather(ci, b):
            for j in range(K):
                g_cp(ci, b, j).wait()

        def fire_scatter(b):
            for j in range(K):
                s_cp(b, j).start(add=True)

        def wait_scatter(b):
            for j in range(K):
                s_cp(b, j).wait()

        d_cp(0, 0).start()
        fire_gather(0, 0)
        d_cp(1, 1).start()
        fire_gather(1, 1)
        wait_gather(0, 0)
        d_cp(0, 0).wait()
        fire_scatter(0)

        def body(tt, carry):
            for b in (0, 1):
                i = 2 * tt + b
                wait_scatter(b)
                d_cp(i, b).start()
                fire_gather(i, b)
                wait_gather(i - 1, 1 - b)
                d_cp(i - 1, 1 - b).wait()
                fire_scatter(1 - b)
            return carry

        lax.fori_loop(1, n_iter // 2, body, 0)

        last = n_iter - 1
        wait_gather(last, 1)
        d_cp(last, 1).wait()
        fire_scatter(1)
        wait_scatter(0)
        wait_scatter(1)

        plsc.subcore_barrier()
        pltpu.sync_copy(acc_sh.at[pl.ds(sid * rows_per_sub, rows_per_sub)],
                        out_hbm.at[cid].at[pl.ds(sid * rows_per_sub, rows_per_sub)])

    return k(h, src2d, dst2d, zero_rows)


def _sc_edge_scatter_pair(h2, src2d, dst2d, zero_rows, D, K=2):
    mesh = plsc.VectorSubcoreMesh(core_axis_name="c", subcore_axis_name="s")
    n_iter = CHUNKS_PER_S // K
    assert n_iter % 2 == 0 and n_iter >= 4
    rows_per_sub = N_PAD // NS

    @functools.partial(
        pl.kernel,
        out_type=jax.ShapeDtypeStruct((NC, N_PAD, D), jnp.float32),
        mesh=mesh,
        scratch_types=[
            pltpu.VMEM((CHUNKS_PER_S, 128), jnp.int32),
            pltpu.VMEM((K, 128), jnp.int32),
            pltpu.VMEM((K, 128), jnp.int32),
            pltpu.VMEM((K, 128, D), jnp.float32),
            pltpu.VMEM((K, 128, D), jnp.float32),
            pltpu.VMEM_SHARED((N_PAD, D), jnp.float32),
            pltpu.SemaphoreType.DMA,
            pltpu.SemaphoreType.DMA,
            pltpu.SemaphoreType.DMA,
            pltpu.SemaphoreType.DMA,
            pltpu.SemaphoreType.DMA,
            pltpu.SemaphoreType.DMA,
        ],
        compiler_params=pltpu.CompilerParams(use_tc_tiling_on_sc=False),
    )
    def k(hl_hbm, hr_hbm, src_hbm, dst_hbm, zero_hbm, out_hbm,
          src_all, dst0, dst1, rows0, rows1, acc_sh,
          gs0, gs1, ss0, ss1, is0, is1):
        cid = lax.axis_index("c")
        sid = lax.axis_index("s")
        rows = (rows0, rows1)
        dstv = (dst0, dst1)
        gsem = (gs0, gs1)
        ssem = (ss0, ss1)
        isem = (is0, is1)

        pltpu.sync_copy(src_hbm.at[pl.ds(sid * CHUNKS_PER_S, CHUNKS_PER_S)], src_all)
        pltpu.sync_copy(zero_hbm.at[pl.ds(sid * rows_per_sub, rows_per_sub)],
                        acc_sh.at[pl.ds(sid * rows_per_sub, rows_per_sub)])
        plsc.subcore_barrier()

        def d_cp(ci, b):
            base = sid * CHUNKS_PER_S + ci * K
            return pltpu.make_async_copy(dst_hbm.at[pl.ds(base, K)], dstv[b], isem[b])

        def s_cp(b, j):
            return pltpu.make_async_copy(
                rows[b].at[j], acc_sh.at[dstv[b].at[j]], ssem[b])

        def fire_scatter(b):
            for j in range(K):
                s_cp(b, j).start(add=True)

        def wait_scatter(b):
            for j in range(K):
                s_cp(b, j).wait()

        def run(h_hbm):
            def g_cp(ci, b, j):
                return pltpu.make_async_copy(
                    h_hbm.at[src_all.at[ci * K + j]], rows[b].at[j], gsem[b])

            def fire_gather(ci, b):
                for j in range(K):
                    g_cp(ci, b, j).start()

            def wait_gather(ci, b):
                for j in range(K):
                    g_cp(ci, b, j).wait()

            d_cp(0, 0).start()
            fire_gather(0, 0)
            d_cp(1, 1).start()
            fire_gather(1, 1)
            wait_gather(0, 0)
            d_cp(0, 0).wait()
            fire_scatter(0)

            def body(tt, carry):
                for b in (0, 1):
                    i = 2 * tt + b
                    wait_scatter(b)
                    d_cp(i, b).start()
                    fire_gather(i, b)
                    wait_gather(i - 1, 1 - b)
                    d_cp(i - 1, 1 - b).wait()
                    fire_scatter(1 - b)
                return carry

            lax.fori_loop(1, n_iter // 2, body, 0)

            last = n_iter - 1
            wait_gather(last, 1)
            d_cp(last, 1).wait()
            fire_scatter(1)
            wait_scatter(0)
            wait_scatter(1)

        @pl.when(cid == 0)
        def _():
            run(hl_hbm)

        @pl.when(cid == 1)
        def _():
            run(hr_hbm)

        plsc.subcore_barrier()
        pltpu.sync_copy(acc_sh.at[pl.ds(sid * rows_per_sub, rows_per_sub)],
                        out_hbm.at[cid].at[pl.ds(sid * rows_per_sub, rows_per_sub)])

    return k(h2[0], h2[1], src2d, dst2d, zero_rows)


def _sc_degree(dst2d, zero_col, ones128):
    mesh = plsc.VectorSubcoreMesh(core_axis_name="c", subcore_axis_name="s")
    K = 8
    n_iter = CHUNKS_PER_W // K
    rows_per_sub = N_PAD // NS

    @functools.partial(
        pl.kernel,
        out_type=jax.ShapeDtypeStruct((NC, N_PAD, 16), jnp.float32),
        mesh=mesh,
        scratch_types=[
            pltpu.VMEM((K, 128), jnp.int32),
            pltpu.VMEM((K, 128), jnp.int32),
            pltpu.VMEM((128, 16), jnp.float32),
            pltpu.VMEM_SHARED((N_PAD, 16), jnp.float32),
            pltpu.SemaphoreType.DMA,
            pltpu.SemaphoreType.DMA,
            pltpu.SemaphoreType.DMA,
            pltpu.SemaphoreType.DMA,
        ],
        compiler_params=pltpu.CompilerParams(use_tc_tiling_on_sc=False),
    )
    def k(dst_hbm, zero_hbm, ones_hbm, out_hbm, dst0, dst1, ones_v, acc_sh,
          ss0, ss1, is0, is1):
        cid = lax.axis_index("c")
        sid = lax.axis_index("s")
        wid = sid * NC + cid
        dstv = (dst0, dst1)
        ssem = (ss0, ss1)
        isem = (is0, is1)
        pltpu.sync_copy(ones_hbm, ones_v)
        pltpu.sync_copy(zero_hbm.at[pl.ds(sid * rows_per_sub, rows_per_sub)],
                        acc_sh.at[pl.ds(sid * rows_per_sub, rows_per_sub)])
        plsc.subcore_barrier()

        def d_cp(ci, b):
            base = wid * CHUNKS_PER_W + ci * K
            return pltpu.make_async_copy(dst_hbm.at[pl.ds(base, K)], dstv[b], isem[b])

        def s_cp(b, j):
            return pltpu.make_async_copy(ones_v, acc_sh.at[dstv[b].at[j]], ssem[b])

        def fire_scatter(b):
            for j in range(K):
                s_cp(b, j).start(add=True)

        def wait_scatter(b):
            for j in range(K):
                s_cp(b, j).wait()

        d_cp(0, 0).start()
        d_cp(1, 1).start()
        d_cp(0, 0).wait()
        fire_scatter(0)

        def body(tt, carry):
            for b in (0, 1):
                i = 2 * tt + b
                wait_scatter(b)
                d_cp(i, b).start()
                d_cp(i - 1, 1 - b).wait()
                fire_scatter(1 - b)
            return carry

        lax.fori_loop(1, n_iter // 2, body, 0)

        last = n_iter - 1
        d_cp(last, 1).wait()
        fire_scatter(1)
        wait_scatter(0)
        wait_scatter(1)

        plsc.subcore_barrier()
        pltpu.sync_copy(acc_sh.at[pl.ds(sid * rows_per_sub, rows_per_sub)],
                        out_hbm.at[cid].at[pl.ds(sid * rows_per_sub, rows_per_sub)])

    return k(dst2d, zero_col, ones128)


_BR = 1024


def _tc_dinv_scale(degpart, t):
    D = t.shape[1]
    grid = (N_PAD // _BR,)

    def body(d_ref, t_ref, s_ref, o_ref):
        d = d_ref[0] + d_ref[1] + 1.0
        s = lax.rsqrt(d)
        s_ref[...] = s
        o_ref[...] = t_ref[...] * s

    return pl.pallas_call(
        body,
        grid=grid,
        in_specs=[
            pl.BlockSpec((2, _BR, 1), lambda i: (0, i, 0)),
            pl.BlockSpec((_BR, D), lambda i: (i, 0)),
        ],
        out_specs=[
            pl.BlockSpec((_BR, 1), lambda i: (i, 0)),
            pl.BlockSpec((_BR, D), lambda i: (i, 0)),
        ],
        out_shape=[
            jax.ShapeDtypeStruct((N_PAD, 1), jnp.float32),
            jax.ShapeDtypeStruct((N_PAD, D), jnp.float32),
        ],
    )(degpart, t)


def _tc_mm(x, w, b, scale=None, relu=False):
    Din, Dout = w.shape
    grid = (N_PAD // _BR,)
    in_specs = [
        pl.BlockSpec((_BR, Din), lambda i: (i, 0)),
        pl.BlockSpec((Din, Dout), lambda i: (0, 0)),
        pl.BlockSpec((1, Dout), lambda i: (0, 0)),
    ]
    args = [x, w, b.reshape(1, Dout)]
    if scale is not None:
        in_specs.append(pl.BlockSpec((_BR, 1), lambda i: (i, 0)))
        args.append(scale)

    def body(x_ref, w_ref, b_ref, *rest):
        o_ref = rest[-1]
        y = jnp.dot(x_ref[...], w_ref[...], preferred_element_type=jnp.float32)
        y = y + b_ref[...]
        if scale is not None:
            y = y * rest[0][...]
        if relu:
            y = jnp.maximum(y, 0.0)
        o_ref[...] = y

    return pl.pallas_call(
        body,
        grid=grid,
        in_specs=in_specs,
        out_specs=pl.BlockSpec((_BR, Dout), lambda i: (i, 0)),
        out_shape=jax.ShapeDtypeStruct((N_PAD, Dout), jnp.float32),
    )(*args)


def _tc_stage(acc, h, dinv, b, w, b2, scale_out=True, in_halves=False,
              out_split=0, zero_tail=False):
    Dx = acc.shape[2] * (2 if in_halves else 1)
    Din, Dout = w.shape
    assert Din == Dx
    grid = (N_PAD // _BR,)
    Dh = acc.shape[2]

    def body(a_ref, h_ref, s_ref, b_ref, w_ref, b2_ref, o_ref):
        if in_halves:
            t = (jnp.concatenate([a_ref[0], a_ref[1]], axis=1)
                 + jnp.concatenate([h_ref[0], h_ref[1]], axis=1))
        else:
            t = a_ref[0] + a_ref[1] + h_ref[...]
        s = s_ref[...]
        x = jnp.maximum(t * s + b_ref[...], 0.0)
        y = jnp.dot(x, w_ref[...], preferred_element_type=jnp.float32)
        y = y + b2_ref[...]
        if scale_out:
            y = y * s
        if zero_tail:
            row = (pl.program_id(0) * _BR
                   + lax.broadcasted_iota(jnp.int32, (_BR, 1), 0))
            y = jnp.where(row < N, y, 0.0)
        if out_split:
            o_ref[0] = y[:, :out_split]
            o_ref[1] = y[:, out_split:]
        else:
            o_ref[...] = y

    h_spec = (pl.BlockSpec((2, _BR, Dh), lambda i: (0, i, 0)) if in_halves
              else pl.BlockSpec((_BR, Dx), lambda i: (i, 0)))
    if out_split:
        out_spec = pl.BlockSpec((2, _BR, out_split), lambda i: (0, i, 0))
        out_shape = jax.ShapeDtypeStruct((2, N_PAD, out_split), jnp.float32)
    else:
        out_spec = pl.BlockSpec((_BR, Dout), lambda i: (i, 0))
        out_shape = jax.ShapeDtypeStruct((N_PAD, Dout), jnp.float32)

    return pl.pallas_call(
        body,
        grid=grid,
        in_specs=[
            pl.BlockSpec((2, _BR, Dh), lambda i: (0, i, 0)),
            h_spec,
            pl.BlockSpec((_BR, 1), lambda i: (i, 0)),
            pl.BlockSpec((1, Dx), lambda i: (0, 0)),
            pl.BlockSpec((Din, Dout), lambda i: (0, 0)),
            pl.BlockSpec((1, Dout), lambda i: (0, 0)),
        ],
        out_specs=out_spec,
        out_shape=out_shape,
    )(acc, h, dinv, b.reshape(1, Dx), w, b2.reshape(1, Dout))


def _tc_post(acc, h, dinv, b, in_halves=False):
    Dh = acc.shape[2]
    D = Dh * (2 if in_halves else 1)
    grid = (N_PAD // _BR,)

    def body(a_ref, h_ref, s_ref, b_ref, o_ref):
        if in_halves:
            t = (jnp.concatenate([a_ref[0], a_ref[1]], axis=1)
                 + jnp.concatenate([h_ref[0], h_ref[1]], axis=1))
        else:
            t = a_ref[0] + a_ref[1] + h_ref[...]
        o_ref[...] = jnp.maximum(t * s_ref[...] + b_ref[...], 0.0)

    h_spec = (pl.BlockSpec((2, _BR, Dh), lambda i: (0, i, 0)) if in_halves
              else pl.BlockSpec((_BR, D), lambda i: (i, 0)))
    return pl.pallas_call(
        body,
        grid=grid,
        in_specs=[
            pl.BlockSpec((2, _BR, Dh), lambda i: (0, i, 0)),
            h_spec,
            pl.BlockSpec((_BR, 1), lambda i: (i, 0)),
            pl.BlockSpec((1, D), lambda i: (0, 0)),
        ],
        out_specs=pl.BlockSpec((_BR, D), lambda i: (i, 0)),
        out_shape=jax.ShapeDtypeStruct((N_PAD, D), jnp.float32),
    )(acc, h, dinv, b.reshape(1, D))


def _tc_projproj(a, w1, b1, w2, dinv):
    D1 = w1.shape[1]
    D2 = w2.shape[1]
    grid = (N_PAD // _BR,)

    def body(a_ref, w1_ref, b1_ref, w2_ref, s_ref, o_ref):
        p = jnp.dot(a_ref[...], w1_ref[...], preferred_element_type=jnp.float32)
        p = p + b1_ref[...]
        y = jnp.dot(p, w2_ref[...], preferred_element_type=jnp.float32)
        o_ref[...] = y * s_ref[...]

    return pl.pallas_call(
        body,
        grid=grid,
        in_specs=[
            pl.BlockSpec((_BR, a.shape[1]), lambda i: (i, 0)),
            pl.BlockSpec(w1.shape, lambda i: (0, 0)),
            pl.BlockSpec((1, D1), lambda i: (0, 0)),
            pl.BlockSpec(w2.shape, lambda i: (0, 0)),
            pl.BlockSpec((_BR, 1), lambda i: (i, 0)),
        ],
        out_specs=pl.BlockSpec((_BR, D2), lambda i: (i, 0)),
        out_shape=jax.ShapeDtypeStruct((N_PAD, D2), jnp.float32),
    )(a, w1, b1.reshape(1, D1), w2, dinv)


def _tc_attention(q, k, v):
    BQ = 512
    Dh = q.shape[1]
    scale = 1.0 / (Dh ** 0.5)
    grid = (N_PAD // BQ,)

    def body(q_ref, k_ref, v_ref, o_ref):
        s = lax.dot_general(q_ref[...], k_ref[...],
                            (((1,), (1,)), ((), ())),
                            preferred_element_type=jnp.float32) * scale
        m = jnp.max(s, axis=1, keepdims=True)
        p = jnp.exp(s - m)
        l = jnp.sum(p, axis=1, keepdims=True)
        l = l - (N_PAD - N) * jnp.exp(-m)
        pv = jnp.dot(p.astype(jnp.bfloat16), v_ref[...].astype(jnp.bfloat16),
                     preferred_element_type=jnp.float32)
        o_ref[...] = pv / l

    return pl.pallas_call(
        body,
        grid=grid,
        in_specs=[
            pl.BlockSpec((BQ, Dh), lambda i: (i, 0)),
            pl.BlockSpec((N_PAD, Dh), lambda i: (0, 0)),
            pl.BlockSpec((N_PAD, Dh), lambda i: (0, 0)),
        ],
        out_specs=pl.BlockSpec((BQ, Dh), lambda i: (i, 0)),
        out_shape=jax.ShapeDtypeStruct((N_PAD, Dh), jnp.float32),
    )(q, k, v)


def _block_diag(a, b):
    r1, c1 = a.shape
    r2, c2 = b.shape
    top = jnp.concatenate([a, jnp.zeros((r1, c2), a.dtype)], axis=1)
    bot = jnp.concatenate([jnp.zeros((r2, c1), b.dtype), b], axis=1)
    return jnp.concatenate([top, bot], axis=0)


def kernel(adj, features, W1, b1, W2, b2, W3, b3, attn_in_w, attn_in_b,
           attn_out_w, attn_out_b, W4, b4, W5a, b5a, W6a, b6a, W7a, b7a,
           W5f, b5f, W6f, b6f, W7f, b7f):
    f32 = jnp.float32

    x0 = jnp.pad(features, ((0, N_PAD - N), (0, 0)))
    pad_idx = (N + (jnp.arange(E_PAD - E, dtype=jnp.int32) % (N_PAD - N))).astype(jnp.int32)
    src2d = jnp.concatenate([adj[0], pad_idx]).reshape(ROWS128, 128)
    dst2d = jnp.concatenate([adj[1], pad_idx]).reshape(ROWS128, 128)

    zero_cache = {}

    def zrows(D):
        if D not in zero_cache:
            zero_cache[D] = jnp.zeros((N_PAD, D), f32)
        return zero_cache[D]

    ones128 = jnp.ones((128, 16), f32)
    z32 = jnp.zeros((32,), f32)
    z64 = jnp.zeros((64,), f32)

    degp = _sc_degree(dst2d, zrows(16), ones128)
    t1 = _tc_mm(x0, W1, z64)
    dinv, h = _tc_dinv_scale(degp[:, :, :1], t1)
    acc = _sc_edge_scatter(h, src2d, dst2d, zrows(64), 64)
    h = _tc_stage(acc, h, dinv, b1, W2, z32)
    acc = _sc_edge_scatter(h, src2d, dst2d, zrows(32), 32)
    h = _tc_stage(acc, h, dinv, b2, W3, z32)
    acc = _sc_edge_scatter(h, src2d, dst2d, zrows(32), 32)
    qkv = _tc_stage(acc, h, dinv, b3, attn_in_w.T, attn_in_b, scale_out=False,
                    zero_tail=True)

    a = _tc_attention(qkv[:, :32], qkv[:, 32:64], qkv[:, 64:])
    h = _tc_projproj(a, attn_out_w.T, attn_out_b, W4, dinv)
    acc = _sc_edge_scatter(h, src2d, dst2d, zrows(32), 32)

    W5 = jnp.concatenate([W5a, W5f], axis=1)
    b5 = jnp.concatenate([b5a, b5f])
    h = _tc_stage(acc, h, dinv, b4, W5, z64)
    acc = _sc_edge_scatter(h, src2d, dst2d, zrows(64), 64)

    W6 = _block_diag(W6a, W6f)
    b6 = jnp.concatenate([b6a, b6f])
    h = _tc_stage(acc, h, dinv, b5, W6, jnp.zeros((128,), f32), out_split=64)
    acc = _sc_edge_scatter_pair(h, src2d, dst2d, zrows(64), 64)

    W7 = jnp.pad(_block_diag(W7a, W7f), ((0, 0), (0, 30)))
    b7 = jnp.pad(jnp.concatenate([b7a, b7f]), (0, 30))
    h = _tc_stage(acc, h, dinv, b6, W7, jnp.zeros((160,), f32),
                  in_halves=True, out_split=80)
    acc = _sc_edge_scatter_pair(h, src2d, dst2d, zrows(80), 80)

    out = _tc_post(acc, h, dinv, b7, in_halves=True)
    da = out[:N, :2]
    df = out[:N, 2:130]
    return (da, df)


# speedup vs baseline: 23.8989x; 1.0054x over previous
import functools

import jax
import jax.numpy as jnp
from jax import lax
from jax.experimental import pallas as pl
from jax.experimental.pallas import tpu as pltpu
from jax.experimental.pallas import tpu_sc as plsc

N = 10000
N_PAD = 10240
E = 320000
E_PAD = 327680
ROWS128 = E_PAD // 128
NC, NS = 2, 16
NW = NC * NS
CHUNKS_PER_W = ROWS128 // NW
CHUNKS_PER_S = ROWS128 // NS


def _sc_edge_scatter(h, src2d, dst2d, zero_rows, D, K=4):
    mesh = plsc.VectorSubcoreMesh(core_axis_name="c", subcore_axis_name="s")
    n_iter = CHUNKS_PER_W // K
    assert n_iter % 2 == 0 and n_iter >= 4
    rows_per_sub = N_PAD // NS

    @functools.partial(
        pl.kernel,
        out_type=jax.ShapeDtypeStruct((NC, N_PAD, D), jnp.float32),
        mesh=mesh,
        scratch_types=[
            pltpu.VMEM((CHUNKS_PER_W, 128), jnp.int32),
            pltpu.VMEM((K, 128), jnp.int32),
            pltpu.VMEM((K, 128), jnp.int32),
            pltpu.VMEM((K, 128, D), jnp.float32),
            pltpu.VMEM((K, 128, D), jnp.float32),
            pltpu.VMEM_SHARED((N_PAD, D), jnp.float32),
            pltpu.SemaphoreType.DMA,
            pltpu.SemaphoreType.DMA,
            pltpu.SemaphoreType.DMA,
            pltpu.SemaphoreType.DMA,
            pltpu.SemaphoreType.DMA,
            pltpu.SemaphoreType.DMA,
        ],
        compiler_params=pltpu.CompilerParams(use_tc_tiling_on_sc=False),
    )
    def k(h_hbm, src_hbm, dst_hbm, zero_hbm, out_hbm,
          src_all, dst0, dst1, rows0, rows1, acc_sh,
          gs0, gs1, ss0, ss1, is0, is1):
        cid = lax.axis_index("c")
        sid = lax.axis_index("s")
        wid = sid * NC + cid
        rows = (rows0, rows1)
        dstv = (dst0, dst1)
        gsem = (gs0, gs1)
        ssem = (ss0, ss1)
        isem = (is0, is1)

        pltpu.sync_copy(src_hbm.at[pl.ds(wid * CHUNKS_PER_W, CHUNKS_PER_W)], src_all)
        pltpu.sync_copy(zero_hbm.at[pl.ds(sid * rows_per_sub, rows_per_sub)],
                        acc_sh.at[pl.ds(sid * rows_per_sub, rows_per_sub)])
        plsc.subcore_barrier()

        def d_cp(ci, b):
            base = wid * CHUNKS_PER_W + ci * K
            return pltpu.make_async_copy(dst_hbm.at[pl.ds(base, K)], dstv[b], isem[b])

        def g_cp(ci, b, j):
            return pltpu.make_async_copy(
                h_hbm.at[src_all.at[ci * K + j]], rows[b].at[j], gsem[b])

        def s_cp(b, j):
            return pltpu.make_async_copy(
                rows[b].at[j], acc_sh.at[dstv[b].at[j]], ssem[b])

        def fire_gather(ci, b):
            for j in range(K):
                g_cp(ci, b, j).start()

        def wait_gather(ci, b):
            for j in range(K):
                g_cp(ci, b, j).wait()

        def fire_scatter(b):
            for j in range(K):
                s_cp(b, j).start(add=True)

        def wait_scatter(b):
            for j in range(K):
                s_cp(b, j).wait()

        d_cp(0, 0).start()
        fire_gather(0, 0)
        d_cp(1, 1).start()
        fire_gather(1, 1)
        wait_gather(0, 0)
        d_cp(0, 0).wait()
        fire_scatter(0)

        def body(tt, carry):
            for b in (0, 1):
                i = 2 * tt + b
                wait_scatter(b)
                d_cp(i, b).start()
                fire_gather(i, b)
                wait_gather(i - 1, 1 - b)
                d_cp(i - 1, 1 - b).wait()
                fire_scatter(1 - b)
            return carry

        lax.fori_loop(1, n_iter // 2, body, 0)

        last = n_iter - 1
        wait_gather(last, 1)
        d_cp(last, 1).wait()
        fire_scatter(1)
        wait_scatter(0)
        wait_scatter(1)

        plsc.subcore_barrier()
        pltpu.sync_copy(acc_sh.at[pl.ds(sid * rows_per_sub, rows_per_sub)],
                        out_hbm.at[cid].at[pl.ds(sid * rows_per_sub, rows_per_sub)])

    return k(h, src2d, dst2d, zero_rows)


def _sc_edge_scatter_pair(h2, src2d, dst2d, zero_rows, D, K=2):
    mesh = plsc.VectorSubcoreMesh(core_axis_name="c", subcore_axis_name="s")
    n_iter = CHUNKS_PER_S // K
    assert n_iter % 2 == 0 and n_iter >= 4
    rows_per_sub = N_PAD // NS

    @functools.partial(
        pl.kernel,
        out_type=jax.ShapeDtypeStruct((NC, N_PAD, D), jnp.float32),
        mesh=mesh,
        scratch_types=[
            pltpu.VMEM((CHUNKS_PER_S, 128), jnp.int32),
            pltpu.VMEM((K, 128), jnp.int32),
            pltpu.VMEM((K, 128), jnp.int32),
            pltpu.VMEM((K, 128, D), jnp.float32),
            pltpu.VMEM((K, 128, D), jnp.float32),
            pltpu.VMEM_SHARED((N_PAD, D), jnp.float32),
            pltpu.SemaphoreType.DMA,
            pltpu.SemaphoreType.DMA,
            pltpu.SemaphoreType.DMA,
            pltpu.SemaphoreType.DMA,
            pltpu.SemaphoreType.DMA,
            pltpu.SemaphoreType.DMA,
        ],
        compiler_params=pltpu.CompilerParams(use_tc_tiling_on_sc=False),
    )
    def k(hl_hbm, hr_hbm, src_hbm, dst_hbm, zero_hbm, out_hbm,
          src_all, dst0, dst1, rows0, rows1, acc_sh,
          gs0, gs1, ss0, ss1, is0, is1):
        cid = lax.axis_index("c")
        sid = lax.axis_index("s")
        rows = (rows0, rows1)
        dstv = (dst0, dst1)
        gsem = (gs0, gs1)
        ssem = (ss0, ss1)
        isem = (is0, is1)

        pltpu.sync_copy(src_hbm.at[pl.ds(sid * CHUNKS_PER_S, CHUNKS_PER_S)], src_all)
        pltpu.sync_copy(zero_hbm.at[pl.ds(sid * rows_per_sub, rows_per_sub)],
                        acc_sh.at[pl.ds(sid * rows_per_sub, rows_per_sub)])
        plsc.subcore_barrier()

        def d_cp(ci, b):
            base = sid * CHUNKS_PER_S + ci * K
            return pltpu.make_async_copy(dst_hbm.at[pl.ds(base, K)], dstv[b], isem[b])

        def s_cp(b, j):
            return pltpu.make_async_copy(
                rows[b].at[j], acc_sh.at[dstv[b].at[j]], ssem[b])

        def fire_scatter(b):
            for j in range(K):
                s_cp(b, j).start(add=True)

        def wait_scatter(b):
            for j in range(K):
                s_cp(b, j).wait()

        def run(h_hbm):
            def g_cp(ci, b, j):
                return pltpu.make_async_copy(
                    h_hbm.at[src_all.at[ci * K + j]], rows[b].at[j], gsem[b])

            def fire_gather(ci, b):
                for j in range(K):
                    g_cp(ci, b, j).start()

            def wait_gather(ci, b):
                for j in range(K):
                    g_cp(ci, b, j).wait()

            d_cp(0, 0).start()
            fire_gather(0, 0)
            d_cp(1, 1).start()
            fire_gather(1, 1)
            wait_gather(0, 0)
            d_cp(0, 0).wait()
            fire_scatter(0)

            def body(tt, carry):
                for b in (0, 1):
                    i = 2 * tt + b
                    wait_scatter(b)
                    d_cp(i, b).start()
                    fire_gather(i, b)
                    wait_gather(i - 1, 1 - b)
                    d_cp(i - 1, 1 - b).wait()
                    fire_scatter(1 - b)
                return carry

            lax.fori_loop(1, n_iter // 2, body, 0)

            last = n_iter - 1
            wait_gather(last, 1)
            d_cp(last, 1).wait()
            fire_scatter(1)
            wait_scatter(0)
            wait_scatter(1)

        @pl.when(cid == 0)
        def _():
            run(hl_hbm)

        @pl.when(cid == 1)
        def _():
            run(hr_hbm)

        plsc.subcore_barrier()
        pltpu.sync_copy(acc_sh.at[pl.ds(sid * rows_per_sub, rows_per_sub)],
                        out_hbm.at[cid].at[pl.ds(sid * rows_per_sub, rows_per_sub)])

    return k(h2[0], h2[1], src2d, dst2d, zero_rows)


def _sc_degree(dst2d, zero_col, ones128):
    mesh = plsc.VectorSubcoreMesh(core_axis_name="c", subcore_axis_name="s")
    K = 8
    n_iter = CHUNKS_PER_W // K
    rows_per_sub = N_PAD // NS

    @functools.partial(
        pl.kernel,
        out_type=jax.ShapeDtypeStruct((NC, N_PAD, 16), jnp.float32),
        mesh=mesh,
        scratch_types=[
            pltpu.VMEM((K, 128), jnp.int32),
            pltpu.VMEM((K, 128), jnp.int32),
            pltpu.VMEM((128, 16), jnp.float32),
            pltpu.VMEM_SHARED((N_PAD, 16), jnp.float32),
            pltpu.SemaphoreType.DMA,
            pltpu.SemaphoreType.DMA,
            pltpu.SemaphoreType.DMA,
            pltpu.SemaphoreType.DMA,
        ],
        compiler_params=pltpu.CompilerParams(use_tc_tiling_on_sc=False),
    )
    def k(dst_hbm, zero_hbm, ones_hbm, out_hbm, dst0, dst1, ones_v, acc_sh,
          ss0, ss1, is0, is1):
        cid = lax.axis_index("c")
        sid = lax.axis_index("s")
        wid = sid * NC + cid
        dstv = (dst0, dst1)
        ssem = (ss0, ss1)
        isem = (is0, is1)
        pltpu.sync_copy(ones_hbm, ones_v)
        pltpu.sync_copy(zero_hbm.at[pl.ds(sid * rows_per_sub, rows_per_sub)],
                        acc_sh.at[pl.ds(sid * rows_per_sub, rows_per_sub)])
        plsc.subcore_barrier()

        def d_cp(ci, b):
            base = wid * CHUNKS_PER_W + ci * K
            return pltpu.make_async_copy(dst_hbm.at[pl.ds(base, K)], dstv[b], isem[b])

        def s_cp(b, j):
            return pltpu.make_async_copy(ones_v, acc_sh.at[dstv[b].at[j]], ssem[b])

        def fire_scatter(b):
            for j in range(K):
                s_cp(b, j).start(add=True)

        def wait_scatter(b):
            for j in range(K):
                s_cp(b, j).wait()

        d_cp(0, 0).start()
        d_cp(1, 1).start()
        d_cp(0, 0).wait()
        fire_scatter(0)

        def body(tt, carry):
            for b in (0, 1):
                i = 2 * tt + b
                wait_scatter(b)
                d_cp(i, b).start()
                d_cp(i - 1, 1 - b).wait()
                fire_scatter(1 - b)
            return carry

        lax.fori_loop(1, n_iter // 2, body, 0)

        last = n_iter - 1
        d_cp(last, 1).wait()
        fire_scatter(1)
        wait_scatter(0)
        wait_scatter(1)

        plsc.subcore_barrier()
        pltpu.sync_copy(acc_sh.at[pl.ds(sid * rows_per_sub, rows_per_sub)],
                        out_hbm.at[cid].at[pl.ds(sid * rows_per_sub, rows_per_sub)])

    return k(dst2d, zero_col, ones128)


_BR = 1024


def _tc_dinv_scale(degpart, t):
    D = t.shape[1]
    grid = (N_PAD // _BR,)

    def body(d_ref, t_ref, s_ref, o_ref):
        d = d_ref[0] + d_ref[1] + 1.0
        s = lax.rsqrt(d)
        s_ref[...] = s
        o_ref[...] = t_ref[...] * s

    return pl.pallas_call(
        body,
        grid=grid,
        in_specs=[
            pl.BlockSpec((2, _BR, 1), lambda i: (0, i, 0)),
            pl.BlockSpec((_BR, D), lambda i: (i, 0)),
        ],
        out_specs=[
            pl.BlockSpec((_BR, 1), lambda i: (i, 0)),
            pl.BlockSpec((_BR, D), lambda i: (i, 0)),
        ],
        out_shape=[
            jax.ShapeDtypeStruct((N_PAD, 1), jnp.float32),
            jax.ShapeDtypeStruct((N_PAD, D), jnp.float32),
        ],
    )(degpart, t)


def _tc_mm(x, w, b, scale=None, relu=False):
    Din, Dout = w.shape
    grid = (N_PAD // _BR,)
    in_specs = [
        pl.BlockSpec((_BR, Din), lambda i: (i, 0)),
        pl.BlockSpec((Din, Dout), lambda i: (0, 0)),
        pl.BlockSpec((1, Dout), lambda i: (0, 0)),
    ]
    args = [x, w, b.reshape(1, Dout)]
    if scale is not None:
        in_specs.append(pl.BlockSpec((_BR, 1), lambda i: (i, 0)))
        args.append(scale)

    def body(x_ref, w_ref, b_ref, *rest):
        o_ref = rest[-1]
        y = jnp.dot(x_ref[...], w_ref[...], preferred_element_type=jnp.float32)
        y = y + b_ref[...]
        if scale is not None:
            y = y * rest[0][...]
        if relu:
            y = jnp.maximum(y, 0.0)
        o_ref[...] = y

    return pl.pallas_call(
        body,
        grid=grid,
        in_specs=in_specs,
        out_specs=pl.BlockSpec((_BR, Dout), lambda i: (i, 0)),
        out_shape=jax.ShapeDtypeStruct((N_PAD, Dout), jnp.float32),
    )(*args)


def _tc_stage(acc, h, dinv, b, w, b2, scale_out=True, in_halves=False,
              out_split=0, zero_tail=False):
    Dx = acc.shape[2] * (2 if in_halves else 1)
    Din, Dout = w.shape
    assert Din == Dx
    grid = (N_PAD // _BR,)
    Dh = acc.shape[2]

    def body(a_ref, h_ref, s_ref, b_ref, w_ref, b2_ref, o_ref):
        if in_halves:
            t = (jnp.concatenate([a_ref[0], a_ref[1]], axis=1)
                 + jnp.concatenate([h_ref[0], h_ref[1]], axis=1))
        else:
            t = a_ref[0] + a_ref[1] + h_ref[...]
        s = s_ref[...]
        x = jnp.maximum(t * s + b_ref[...], 0.0)
        y = jnp.dot(x, w_ref[...], preferred_element_type=jnp.float32)
        y = y + b2_ref[...]
        if scale_out:
            y = y * s
        if zero_tail:
            row = (pl.program_id(0) * _BR
                   + lax.broadcasted_iota(jnp.int32, (_BR, 1), 0))
            y = jnp.where(row < N, y, 0.0)
        if out_split:
            o_ref[0] = y[:, :out_split]
            o_ref[1] = y[:, out_split:]
        else:
            o_ref[...] = y

    h_spec = (pl.BlockSpec((2, _BR, Dh), lambda i: (0, i, 0)) if in_halves
              else pl.BlockSpec((_BR, Dx), lambda i: (i, 0)))
    if out_split:
        out_spec = pl.BlockSpec((2, _BR, out_split), lambda i: (0, i, 0))
        out_shape = jax.ShapeDtypeStruct((2, N_PAD, out_split), jnp.float32)
    else:
        out_spec = pl.BlockSpec((_BR, Dout), lambda i: (i, 0))
        out_shape = jax.ShapeDtypeStruct((N_PAD, Dout), jnp.float32)

    return pl.pallas_call(
        body,
        grid=grid,
        in_specs=[
            pl.BlockSpec((2, _BR, Dh), lambda i: (0, i, 0)),
            h_spec,
            pl.BlockSpec((_BR, 1), lambda i: (i, 0)),
            pl.BlockSpec((1, Dx), lambda i: (0, 0)),
            pl.BlockSpec((Din, Dout), lambda i: (0, 0)),
            pl.BlockSpec((1, Dout), lambda i: (0, 0)),
        ],
        out_specs=out_spec,
        out_shape=out_shape,
    )(acc, h, dinv, b.reshape(1, Dx), w, b2.reshape(1, Dout))


def _tc_post(acc, h, dinv, b, in_halves=False):
    Dh = acc.shape[2]
    D = Dh * (2 if in_halves else 1)
    grid = (N_PAD // _BR,)

    def body(a_ref, h_ref, s_ref, b_ref, o_ref):
        if in_halves:
            t = (jnp.concatenate([a_ref[0], a_ref[1]], axis=1)
                 + jnp.concatenate([h_ref[0], h_ref[1]], axis=1))
        else:
            t = a_ref[0] + a_ref[1] + h_ref[...]
        o_ref[...] = jnp.maximum(t * s_ref[...] + b_ref[...], 0.0)

    h_spec = (pl.BlockSpec((2, _BR, Dh), lambda i: (0, i, 0)) if in_halves
              else pl.BlockSpec((_BR, D), lambda i: (i, 0)))
    return pl.pallas_call(
        body,
        grid=grid,
        in_specs=[
            pl.BlockSpec((2, _BR, Dh), lambda i: (0, i, 0)),
            h_spec,
            pl.BlockSpec((_BR, 1), lambda i: (i, 0)),
            pl.BlockSpec((1, D), lambda i: (0, 0)),
        ],
        out_specs=pl.BlockSpec((_BR, D), lambda i: (i, 0)),
        out_shape=jax.ShapeDtypeStruct((N_PAD, D), jnp.float32),
    )(acc, h, dinv, b.reshape(1, D))


def _tc_projproj(a, w1, b1, w2, dinv):
    D1 = w1.shape[1]
    D2 = w2.shape[1]
    grid = (N_PAD // _BR,)

    def body(a_ref, w1_ref, b1_ref, w2_ref, s_ref, o_ref):
        p = jnp.dot(a_ref[...], w1_ref[...], preferred_element_type=jnp.float32)
        p = p + b1_ref[...]
        y = jnp.dot(p, w2_ref[...], preferred_element_type=jnp.float32)
        o_ref[...] = y * s_ref[...]

    return pl.pallas_call(
        body,
        grid=grid,
        in_specs=[
            pl.BlockSpec((_BR, a.shape[1]), lambda i: (i, 0)),
            pl.BlockSpec(w1.shape, lambda i: (0, 0)),
            pl.BlockSpec((1, D1), lambda i: (0, 0)),
            pl.BlockSpec(w2.shape, lambda i: (0, 0)),
            pl.BlockSpec((_BR, 1), lambda i: (i, 0)),
        ],
        out_specs=pl.BlockSpec((_BR, D2), lambda i: (i, 0)),
        out_shape=jax.ShapeDtypeStruct((N_PAD, D2), jnp.float32),
    )(a, w1, b1.reshape(1, D1), w2, dinv)


def _tc_attention(q, k, v):
    BQ = 512
    Dh = q.shape[1]
    scale = 1.0 / (Dh ** 0.5)
    grid = (N_PAD // BQ,)

    def body(q_ref, k_ref, v_ref, o_ref):
        s = lax.dot_general(q_ref[...], k_ref[...],
                            (((1,), (1,)), ((), ())),
                            preferred_element_type=jnp.float32) * scale
        m = jnp.max(s, axis=1, keepdims=True)
        p = jnp.exp(s - m)
        l = jnp.sum(p, axis=1, keepdims=True)
        l = l - (N_PAD - N) * jnp.exp(-m)
        pv = jnp.dot(p, v_ref[...], preferred_element_type=jnp.float32)
        o_ref[...] = pv / l

    return pl.pallas_call(
        body,
        grid=grid,
        in_specs=[
            pl.BlockSpec((BQ, Dh), lambda i: (i, 0)),
            pl.BlockSpec((N_PAD, Dh), lambda i: (0, 0)),
            pl.BlockSpec((N_PAD, Dh), lambda i: (0, 0)),
        ],
        out_specs=pl.BlockSpec((BQ, Dh), lambda i: (i, 0)),
        out_shape=jax.ShapeDtypeStruct((N_PAD, Dh), jnp.float32),
    )(q, k, v)


def _block_diag(a, b):
    r1, c1 = a.shape
    r2, c2 = b.shape
    top = jnp.concatenate([a, jnp.zeros((r1, c2), a.dtype)], axis=1)
    bot = jnp.concatenate([jnp.zeros((r2, c1), b.dtype), b], axis=1)
    return jnp.concatenate([top, bot], axis=0)


def kernel(adj, features, W1, b1, W2, b2, W3, b3, attn_in_w, attn_in_b,
           attn_out_w, attn_out_b, W4, b4, W5a, b5a, W6a, b6a, W7a, b7a,
           W5f, b5f, W6f, b6f, W7f, b7f):
    f32 = jnp.float32

    x0 = jnp.pad(features, ((0, N_PAD - N), (0, 0)))
    pad_idx = (N + (jnp.arange(E_PAD - E, dtype=jnp.int32) % (N_PAD - N))).astype(jnp.int32)
    src2d = jnp.concatenate([adj[0], pad_idx]).reshape(ROWS128, 128)
    dst2d = jnp.concatenate([adj[1], pad_idx]).reshape(ROWS128, 128)

    zero_cache = {}

    def zrows(D):
        if D not in zero_cache:
            zero_cache[D] = jnp.zeros((N_PAD, D), f32)
        return zero_cache[D]

    ones128 = jnp.ones((128, 16), f32)
    z32 = jnp.zeros((32,), f32)
    z64 = jnp.zeros((64,), f32)

    degp = _sc_degree(dst2d, zrows(16), ones128)
    t1 = _tc_mm(x0, W1, z64)
    dinv, h = _tc_dinv_scale(degp[:, :, :1], t1)
    acc = _sc_edge_scatter(h, src2d, dst2d, zrows(64), 64)
    h = _tc_stage(acc, h, dinv, b1, W2, z32)
    acc = _sc_edge_scatter(h, src2d, dst2d, zrows(32), 32)
    h = _tc_stage(acc, h, dinv, b2, W3, z32)
    acc = _sc_edge_scatter(h, src2d, dst2d, zrows(32), 32)
    qkv = _tc_stage(acc, h, dinv, b3, attn_in_w.T, attn_in_b, scale_out=False,
                    zero_tail=True)

    a = _tc_attention(qkv[:, :32], qkv[:, 32:64], qkv[:, 64:])
    h = _tc_projproj(a, attn_out_w.T, attn_out_b, W4, dinv)
    acc = _sc_edge_scatter(h, src2d, dst2d, zrows(32), 32)

    W5 = jnp.concatenate([W5a, W5f], axis=1)
    b5 = jnp.concatenate([b5a, b5f])
    h = _tc_stage(acc, h, dinv, b4, W5, z64)
    acc = _sc_edge_scatter(h, src2d, dst2d, zrows(64), 64)

    W6 = _block_diag(W6a, W6f)
    b6 = jnp.concatenate([b6a, b6f])
    h = _tc_stage(acc, h, dinv, b5, W6, jnp.zeros((128,), f32), out_split=64)
    acc = _sc_edge_scatter_pair(h, src2d, dst2d, zrows(64), 64)

    W7 = jnp.pad(_block_diag(W7a, W7f), ((0, 0), (0, 30)))
    b7 = jnp.pad(jnp.concatenate([b7a, b7f]), (0, 30))
    h = _tc_stage(acc, h, dinv, b6, W7, jnp.zeros((160,), f32),
                  in_halves=True, out_split=80)
    acc = _sc_edge_scatter_pair(h, src2d, dst2d, zrows(80), 80)

    out = _tc_post(acc, h, dinv, b7, in_halves=True)
    da = out[:N, :2]
    df = out[:N, 2:130]
    return (da, df)
